# baseline (speedup 1.0000x reference)
import functools

import jax
import jax.numpy as jnp
from jax.experimental import pallas as pl


from jax.experimental.pallas import tpu as pltpu


def _gap_kernel(*refs, cb, ns):
    o_ref = refs[-1]
    for k in range(ns):
        o_ref[0, k * cb:(k + 1) * cb] = jnp.sum(refs[k][0], axis=1)


def _head_kernel(g_ref, w1_ref, b1_ref, w2_ref, b2_ref,
                 ev_ref, ei_ref, loss_ref, *, e, k, eps, scale):
    gap = jnp.sum(g_ref[...], axis=2) * scale
    h = jax.lax.dot_general(
        gap, w1_ref[...], (((1,), (1,)), ((), ())),
        preferred_element_type=jnp.float32)
    h = jax.nn.relu(h + b1_ref[...][None, :])
    pre = jax.lax.dot_general(
        h, w2_ref[...], (((1,), (1,)), ((), ())),
        preferred_element_type=jnp.float32)
    pre = pre + b2_ref[...][None, :]

    m = jnp.max(pre, axis=1, keepdims=True)
    ex = jnp.exp(pre - m)
    logits = ex / jnp.sum(ex, axis=1, keepdims=True)

    b = logits.shape[0]
    ids = jax.lax.broadcasted_iota(jnp.int32, (b, e), 1)

    m1 = jnp.max(logits, axis=1, keepdims=True)
    i1 = jnp.min(jnp.where(logits == m1, ids, e), axis=1, keepdims=True)
    masked = jnp.where(ids == i1, -jnp.inf, logits)
    m2 = jnp.max(masked, axis=1, keepdims=True)
    i2 = jnp.min(jnp.where(masked == m2, ids, e), axis=1, keepdims=True)

    vals = jnp.concatenate([m1, m2], axis=1)
    if k < e:
        ev = jnp.exp(vals - m1)
        vals = ev / jnp.sum(ev, axis=1, keepdims=True)
    ev_ref[...] = vals
    ei_ref[...] = jnp.concatenate([i1, i2], axis=1).astype(jnp.int32)

    n = b * e
    mean = jnp.sum(logits) / n
    var = jnp.sum((logits - mean) ** 2) / (n - 1)
    loss_ref[...] = (jnp.sqrt(var) / (mean + eps)).reshape(1, 1)


def kernel(x, W1, b1, W2, b2):
    B, C, H, W = x.shape
    E = W2.shape[0]
    K = 2
    EPS = 1e-10

    NS = 4
    Cb = 8
    while C % (NS * Cb):
        Cb //= 2
    nc = C // (NS * Cb)

    gap3 = pl.pallas_call(
        functools.partial(_gap_kernel, cb=Cb, ns=NS),
        grid=(B, nc),
        in_specs=[
            pl.BlockSpec((1, Cb, H, W),
                         functools.partial(lambda b, c, k: (b, NS * c + k, 0, 0), k=k))
            for k in range(NS)
        ],
        out_specs=pl.BlockSpec((1, NS * Cb, W), lambda b, c: (b, c, 0)),
        out_shape=jax.ShapeDtypeStruct((B, C, W), jnp.float32),
        compiler_params=pltpu.CompilerParams(
            dimension_semantics=("parallel", "parallel")),
    )(*([x] * NS))

    ev, ei, loss = pl.pallas_call(
        functools.partial(_head_kernel, e=E, k=K, eps=EPS,
                          scale=1.0 / (H * W)),
        out_shape=(
            jax.ShapeDtypeStruct((B, K), jnp.float32),
            jax.ShapeDtypeStruct((B, K), jnp.int32),
            jax.ShapeDtypeStruct((1, 1), jnp.float32),
        ),
    )(gap3, W1, b1, W2, b2)
    return ev, ei, loss[0, 0]


# speedup vs baseline: 1.0005x; 1.0005x over previous
import functools

import jax
import jax.numpy as jnp
from jax.experimental import pallas as pl


from jax.experimental.pallas import tpu as pltpu


def _gap_kernel(x_ref, o_ref, buf, sem, *, cs, nc, nbuf, total):
    i = pl.program_id(0)

    def start(chunk, slot):
        b = chunk // nc
        c = chunk - b * nc
        pltpu.make_async_copy(
            x_ref.at[b, pl.ds(c * cs, cs)], buf.at[slot], sem.at[slot]
        ).start()

    @pl.when(i == 0)
    def _prologue():
        for j in range(min(nbuf, total)):
            start(j, j)

    slot = jax.lax.rem(i, nbuf)
    for j in range(nbuf):
        @pl.when(slot == j)
        def _consume(j=j):
            pltpu.make_async_copy(
                x_ref.at[0, pl.ds(0, cs)], buf.at[j], sem.at[j]
            ).wait()
            o_ref[0] = jnp.sum(buf[j], axis=1)

            @pl.when(i + nbuf < total)
            def _start_next():
                start(i + nbuf, j)


def _head_kernel(g_ref, w1_ref, b1_ref, w2_ref, b2_ref,
                 ev_ref, ei_ref, loss_ref, *, e, k, eps, scale):
    gap = jnp.sum(g_ref[...], axis=2) * scale
    h = jax.lax.dot_general(
        gap, w1_ref[...], (((1,), (1,)), ((), ())),
        preferred_element_type=jnp.float32)
    h = jax.nn.relu(h + b1_ref[...][None, :])
    pre = jax.lax.dot_general(
        h, w2_ref[...], (((1,), (1,)), ((), ())),
        preferred_element_type=jnp.float32)
    pre = pre + b2_ref[...][None, :]

    m = jnp.max(pre, axis=1, keepdims=True)
    ex = jnp.exp(pre - m)
    logits = ex / jnp.sum(ex, axis=1, keepdims=True)

    b = logits.shape[0]
    ids = jax.lax.broadcasted_iota(jnp.int32, (b, e), 1)

    m1 = jnp.max(logits, axis=1, keepdims=True)
    i1 = jnp.min(jnp.where(logits == m1, ids, e), axis=1, keepdims=True)
    masked = jnp.where(ids == i1, -jnp.inf, logits)
    m2 = jnp.max(masked, axis=1, keepdims=True)
    i2 = jnp.min(jnp.where(masked == m2, ids, e), axis=1, keepdims=True)

    vals = jnp.concatenate([m1, m2], axis=1)
    if k < e:
        ev = jnp.exp(vals - m1)
        vals = ev / jnp.sum(ev, axis=1, keepdims=True)
    ev_ref[...] = vals
    ei_ref[...] = jnp.concatenate([i1, i2], axis=1).astype(jnp.int32)

    n = b * e
    mean = jnp.sum(logits) / n
    var = jnp.sum((logits - mean) ** 2) / (n - 1)
    loss_ref[...] = (jnp.sqrt(var) / (mean + eps)).reshape(1, 1)


def kernel(x, W1, b1, W2, b2):
    B, C, H, W = x.shape
    E = W2.shape[0]
    K = 2
    EPS = 1e-10

    NBUF = 8
    Cs = 16
    while C % Cs:
        Cs //= 2
    nc = C // Cs
    total = B * nc

    gap3 = pl.pallas_call(
        functools.partial(_gap_kernel, cs=Cs, nc=nc, nbuf=NBUF, total=total),
        grid=(total,),
        in_specs=[pl.BlockSpec(memory_space=pltpu.MemorySpace.HBM)],
        out_specs=pl.BlockSpec((1, Cs, W), lambda i: (i // nc, i % nc, 0)),
        out_shape=jax.ShapeDtypeStruct((B, C, W), jnp.float32),
        scratch_shapes=[
            pltpu.VMEM((NBUF, Cs, H, W), jnp.float32),
            pltpu.SemaphoreType.DMA((NBUF,)),
        ],
    )(x)

    ev, ei, loss = pl.pallas_call(
        functools.partial(_head_kernel, e=E, k=K, eps=EPS,
                          scale=1.0 / (H * W)),
        out_shape=(
            jax.ShapeDtypeStruct((B, K), jnp.float32),
            jax.ShapeDtypeStruct((B, K), jnp.int32),
            jax.ShapeDtypeStruct((1, 1), jnp.float32),
        ),
    )(gap3, W1, b1, W2, b2)
    return ev, ei, loss[0, 0]
